# Initial kernel scaffold; baseline (speedup 1.0000x reference)
import functools

import jax
import jax.numpy as jnp
from jax import lax
from jax.experimental import pallas as pl
from jax.experimental.pallas import tpu as pltpu
from jax.experimental.pallas import tpu_sc as plsc

N = 10000
E = 320000
D = 128
EPS = 1e-5

NW = 32
NSUB = 16
CH = 128
NCH = 80
E_PAD = NW * NCH * CH
N_PAD = 10240

_f32 = jnp.float32
_i32 = jnp.int32


def _stats_body(w_ref, o_ref):
    w = w_ref[...]
    mu = jnp.sum(w) * (1.0 / E)
    var = jnp.sum((w - mu) ** 2) * (1.0 / E)
    ivs = lax.rsqrt(var + EPS)
    o_ref[...] = jnp.concatenate(
        [jnp.full((1, 128), mu, _f32),
         jnp.full((1, 128), ivs, _f32),
         jnp.zeros((6, 128), _f32)], axis=0)


def _edge_stats(edge_weights):
    return pl.pallas_call(
        _stats_body,
        out_shape=jax.ShapeDtypeStruct((8, 128), _f32),
    )(edge_weights.reshape(E // 128, 128))


def _dense_body(p0_ref, p1_ref, x_ref, ws_ref, wh_ref, bs_ref, bh_ref,
                g_ref, b_ref, wfc_ref, bfc_ref, dp_ref, o_ref, pos_ref,
                *, add_x, emit_pos):
    aggr = p0_ref[...] + p1_ref[...]
    wsum = ws_ref[...] + wh_ref[...]
    h = lax.dot_general(aggr, wsum, (((1,), (1,)), ((), ())),
                        preferred_element_type=_f32)
    h = h + (bs_ref[...] + bh_ref[...])
    m = jnp.mean(h, axis=0)
    v = jnp.mean((h - m) ** 2, axis=0)
    hn = g_ref[...] * (h - m) * lax.rsqrt(v + EPS) + b_ref[...]
    out = lax.dot_general(jnp.maximum(hn, 0.0), wfc_ref[...],
                          (((1,), (1,)), ((), ())),
                          preferred_element_type=_f32)
    out = out + bfc_ref[...]
    if add_x:
        out = out + x_ref[...]
    o_ref[...] = out
    if emit_pos:
        pos_ref[...] = dp_ref[1:2, :] + dp_ref[3:4, :]
    else:
        pos_ref[...] = jnp.zeros((1, N_PAD), _f32)


def _dense_layer(p0, p1, x, dp4, Ws, Wh, bs, bh, gamma, beta, Wfc, bfc,
                 add_x, emit_pos):
    body = functools.partial(_dense_body, add_x=add_x, emit_pos=emit_pos)
    out_shape = (jax.ShapeDtypeStruct((N, D), _f32),
                 jax.ShapeDtypeStruct((1, N_PAD), _f32))
    return pl.pallas_call(
        body,
        out_shape=out_shape,
    )(p0, p1, x, Ws, Wh, bs, bh, gamma, beta, Wfc, bfc, dp4)


def _passA_body(dst_hbm, src_hbm, et_hbm, wt_hbm, nt_hbm, stats_hbm,
                delta_hbm, z2_hbm,
                c_hbm, dp_hbm,
                dstb, srcb, etb, wtb, ntb, statsb, deltab, exb, pb, cb,
                denom_sh, pos_sh):
    c = lax.axis_index("c")
    s = lax.axis_index("s")
    wid = c * NSUB + s

    pltpu.sync_copy(dst_hbm.at[wid], dstb)
    pltpu.sync_copy(src_hbm.at[wid], srcb)
    pltpu.sync_copy(et_hbm.at[wid], etb)
    pltpu.sync_copy(wt_hbm.at[wid], wtb)
    pltpu.sync_copy(nt_hbm, ntb)
    pltpu.sync_copy(stats_hbm, statsb)
    pltpu.sync_copy(delta_hbm, deltab)

    @pl.when(s == 0)
    def _():
        pltpu.sync_copy(z2_hbm.at[0], denom_sh)
        pltpu.sync_copy(z2_hbm.at[1], pos_sh)

    plsc.subcore_barrier()

    mu = statsb[0, pl.ds(0, 16)]
    ivs = statsb[1, pl.ds(0, 16)]
    dl = deltab[...]

    def chunk(j, carry):
        def sub(k, carry2):
            sl = pl.ds(k * 16, 16)
            dstv = dstb[j, sl]
            ntv = plsc.load_gather(ntb, [dstv])
            etv = etb[j, sl]
            wv = wtb[j, sl]
            srcv = srcb[j, sl].astype(_f32)
            td = ntv - etv
            ex = jnp.exp(-dl * td)
            exb[j, sl] = ex
            pb[j, sl] = srcv * jnp.exp(jnp.abs(td))
            cb[j, sl] = (wv - mu) * ivs * ex
            return carry2

        lax.fori_loop(0, CH // 16, sub, 0)
        pltpu.sync_copy(exb.at[j], denom_sh.at[dstb.at[j]], add=True)
        pltpu.sync_copy(pb.at[j], pos_sh.at[dstb.at[j]], add=True)
        return carry

    lax.fori_loop(0, NCH, chunk, 0)
    pltpu.sync_copy(cb, c_hbm.at[wid])

    plsc.subcore_barrier()

    @pl.when(s == 0)
    def _():
        pltpu.sync_copy(denom_sh, dp_hbm.at[c, 0])
        pltpu.sync_copy(pos_sh, dp_hbm.at[c, 1])


def _passA(dstp, srcp, etp, wtp, node_time, stats, delta16, z2):
    mesh = plsc.VectorSubcoreMesh(core_axis_name="c", subcore_axis_name="s")
    k = pl.kernel(
        _passA_body,
        out_type=(jax.ShapeDtypeStruct((NW, NCH, CH), _f32),
                  jax.ShapeDtypeStruct((2, 2, N_PAD), _f32)),
        mesh=mesh,
        scratch_types=[
            pltpu.VMEM((NCH, CH), _i32),
            pltpu.VMEM((NCH, CH), _i32),
            pltpu.VMEM((NCH, CH), _f32),
            pltpu.VMEM((NCH, CH), _f32),
            pltpu.VMEM((N,), _f32),
            pltpu.VMEM((8, 128), _f32),
            pltpu.VMEM((16,), _f32),
            pltpu.VMEM((NCH, CH), _f32),
            pltpu.VMEM((NCH, CH), _f32),
            pltpu.VMEM((NCH, CH), _f32),
            pltpu.VMEM_SHARED((N_PAD,), _f32),
            pltpu.VMEM_SHARED((N_PAD,), _f32),
        ],
    )
    return k(dstp, srcp, etp, wtp, node_time, stats, delta16, z2)


def _passB_body(x_hbm, src_hbm, dst_hbm, cin_hbm, dp_hbm, zbig_hbm,
                *rest, with_rec):
    if with_rec:
        (out_hbm, c2out_hbm,
         srcb, dstb, cb, c2b, recb, d0b, rowsb, sem, aggr_sh) = rest
    else:
        (out_hbm,
         srcb, dstb, cb, c2b, recb, d0b, rowsb, sem, aggr_sh) = rest

    c = lax.axis_index("c")
    s = lax.axis_index("s")
    wid = c * NSUB + s

    pltpu.sync_copy(src_hbm.at[wid], srcb)
    pltpu.sync_copy(dst_hbm.at[wid], dstb)
    pltpu.sync_copy(cin_hbm.at[wid], cb)

    if with_rec:
        pltpu.sync_copy(dp_hbm.at[0, 0], d0b)
        pltpu.sync_copy(dp_hbm.at[1, 0], recb)

        def mkrec(k, carry):
            sl = pl.ds(k * 16, 16)
            v = d0b[sl] + recb[sl]
            recb[sl] = 1.0 / (v + 1e-12)
            return carry

        lax.fori_loop(0, N_PAD // 16, mkrec, 0)

        def c2chunk(j, carry):
            def c2sub(k, carry2):
                sl = pl.ds(k * 16, 16)
                dstv = dstb[j, sl]
                rv = plsc.load_gather(recb, [dstv])
                c2b[j, sl] = cb[j, sl] * rv
                return carry2
            lax.fori_loop(0, CH // 16, c2sub, 0)
            return carry

        lax.fori_loop(0, NCH, c2chunk, 0)
        pltpu.sync_copy(c2b, c2out_hbm.at[wid])

    @pl.when(s == 0)
    def _():
        pltpu.sync_copy(zbig_hbm, aggr_sh)

    plsc.subcore_barrier()

    coeff = c2b if with_rec else cb

    def chunk(j, carry):
        pltpu.async_copy(x_hbm.at[srcb.at[j]], rowsb, sem).wait()

        def row(r, carry2):
            cs = plsc.load_gather(
                coeff, [jnp.full((16,), j, _i32), jnp.full((16,), r, _i32)])
            for m in range(8):
                sl = pl.ds(m * 16, 16)
                rowsb[r, sl] = rowsb[r, sl] * cs
            return carry2

        lax.fori_loop(0, CH, row, 0)
        pltpu.sync_copy(rowsb, aggr_sh.at[dstb.at[j]], add=True)
        return carry

    lax.fori_loop(0, NCH, chunk, 0)

    plsc.subcore_barrier()

    @pl.when(s == 0)
    def _():
        pltpu.sync_copy(aggr_sh, out_hbm.at[c])


def _passB(xtab, srcp, dstp, cin, dp, zbig, with_rec):
    mesh = plsc.VectorSubcoreMesh(core_axis_name="c", subcore_axis_name="s")
    if with_rec:
        out_type = (jax.ShapeDtypeStruct((2, N_PAD, D), _f32),
                    jax.ShapeDtypeStruct((NW, NCH, CH), _f32))
    else:
        out_type = (jax.ShapeDtypeStruct((2, N_PAD, D), _f32),)
    body = functools.partial(_passB_body, with_rec=with_rec)
    k = pl.kernel(
        body,
        out_type=out_type,
        mesh=mesh,
        scratch_types=[
            pltpu.VMEM((NCH, CH), _i32),
            pltpu.VMEM((NCH, CH), _i32),
            pltpu.VMEM((NCH, CH), _f32),
            pltpu.VMEM((NCH, CH), _f32),
            pltpu.VMEM((N_PAD,), _f32),
            pltpu.VMEM((N_PAD,), _f32),
            pltpu.VMEM((CH, D), _f32),
            pltpu.SemaphoreType.DMA,
            pltpu.VMEM_SHARED((N_PAD, D), _f32),
        ],
    )
    return k(xtab, srcp, dstp, cin, dp, zbig)


def kernel(x, edge_index, edge_weights, edge_times, node_time, node_ids,
           delta,
           Ws1, bs1, Wh1, bh1, gamma1, beta1, Wfc1, bfc1,
           Ws2, bs2, Wh2, bh2, gamma2, beta2, Wfc2, bfc2):
    dst = edge_index[0].astype(_i32)
    src = edge_index[1].astype(_i32)
    pad = E_PAD - E

    dstp = jnp.concatenate([dst, jnp.zeros((pad,), _i32)]).reshape(NW, NCH, CH)
    srcp = jnp.concatenate([src, jnp.zeros((pad,), _i32)]).reshape(NW, NCH, CH)
    etp = jnp.concatenate(
        [edge_times, jnp.full((pad,), -60.0, _f32)]).reshape(NW, NCH, CH)
    wtp = jnp.concatenate(
        [edge_weights, jnp.zeros((pad,), _f32)]).reshape(NW, NCH, CH)
    delta16 = jnp.full((16,), delta, _f32)
    z2 = jnp.zeros((2, N_PAD), _f32)
    zbig = jnp.zeros((N_PAD, D), _f32)

    stats = _edge_stats(edge_weights)
    c3, dp = _passA(dstp, srcp, etp, wtp, node_time, stats, delta16, z2)

    p1, c23 = _passB(x, srcp, dstp, c3, dp, zbig, with_rec=True)
    x2, _ = _dense_layer(p1[0, :N], p1[1, :N], x, dp.reshape(4, N_PAD),
                         Ws1, Wh1, bs1, bh1, gamma1, beta1, Wfc1, bfc1,
                         add_x=True, emit_pos=False)

    (p2,) = _passB(x2, srcp, dstp, c23, dp, zbig, with_rec=False)
    out, pos2d = _dense_layer(p2[0, :N], p2[1, :N], x2, dp.reshape(4, N_PAD),
                              Ws2, Wh2, bs2, bh2, gamma2, beta2, Wfc2, bfc2,
                              add_x=False, emit_pos=True)

    pos = pos2d.reshape(N_PAD)[:N]
    return out, pos


# speedup vs baseline: 6.5216x; 6.5216x over previous
import functools

import jax
import jax.numpy as jnp
from jax import lax
from jax.experimental import pallas as pl
from jax.experimental.pallas import tpu as pltpu
from jax.experimental.pallas import tpu_sc as plsc

N = 10000
E = 320000
D = 128
EPS = 1e-5

NW = 32
NSUB = 16
CH = 128
NCH = 80
E_PAD = NW * NCH * CH
N_PAD = 10240

_f32 = jnp.float32
_i32 = jnp.int32


def _stats_body(w_ref, o_ref):
    w = w_ref[...]
    mu = jnp.sum(w) * (1.0 / E)
    var = jnp.sum((w - mu) ** 2) * (1.0 / E)
    ivs = lax.rsqrt(var + EPS)
    o_ref[...] = jnp.concatenate(
        [jnp.full((1, 128), mu, _f32),
         jnp.full((1, 128), ivs, _f32),
         jnp.zeros((6, 128), _f32)], axis=0)


def _edge_stats(edge_weights):
    return pl.pallas_call(
        _stats_body,
        out_shape=jax.ShapeDtypeStruct((8, 128), _f32),
    )(edge_weights.reshape(E // 128, 128))


def _dense_body(p00_ref, p01_ref, p10_ref, p11_ref, x_ref, ws_ref, wh_ref,
                bs_ref, bh_ref, g_ref, b_ref, wfc_ref, bfc_ref, dp_ref,
                o_ref, pos_ref, *, add_x, emit_pos):
    a0 = p00_ref[...] + p10_ref[...]
    a1 = p01_ref[...] + p11_ref[...]
    wsum = ws_ref[...] + wh_ref[...]
    h = (lax.dot_general(a0, wsum[:, :D // 2], (((1,), (1,)), ((), ())),
                         precision=lax.Precision.HIGHEST,
                         preferred_element_type=_f32)
         + lax.dot_general(a1, wsum[:, D // 2:], (((1,), (1,)), ((), ())),
                           precision=lax.Precision.HIGHEST,
                           preferred_element_type=_f32))
    h = h + (bs_ref[...] + bh_ref[...])
    m = jnp.mean(h, axis=0)
    v = jnp.mean((h - m) ** 2, axis=0)
    hn = g_ref[...] * (h - m) * lax.rsqrt(v + EPS) + b_ref[...]
    out = lax.dot_general(jnp.maximum(hn, 0.0), wfc_ref[...],
                          (((1,), (1,)), ((), ())),
                          precision=lax.Precision.HIGHEST,
                          preferred_element_type=_f32)
    out = out + bfc_ref[...]
    if add_x:
        out = out + x_ref[...]
    o_ref[...] = out
    if emit_pos:
        pos_ref[...] = dp_ref[1:2, :] + dp_ref[3:4, :]
    else:
        pos_ref[...] = jnp.zeros((1, N_PAD), _f32)


def _dense_layer(p, x, dp4, Ws, Wh, bs, bh, gamma, beta, Wfc, bfc,
                 add_x, emit_pos):
    body = functools.partial(_dense_body, add_x=add_x, emit_pos=emit_pos)
    out_shape = (jax.ShapeDtypeStruct((N, D), _f32),
                 jax.ShapeDtypeStruct((1, N_PAD), _f32))
    return pl.pallas_call(
        body,
        out_shape=out_shape,
    )(p[0, 0, :N], p[0, 1, :N], p[1, 0, :N], p[1, 1, :N],
      x, Ws, Wh, bs, bh, gamma, beta, Wfc, bfc, dp4)


def _passA_body(dst_hbm, src_hbm, et_hbm, wt_hbm, nt_hbm, stats_hbm,
                delta_hbm, z2_hbm,
                c_hbm, dp_hbm,
                dstb, srcb, etb, wtb, ntb, statsb, deltab, exb, pb, cb,
                denom_sh, pos_sh):
    c = lax.axis_index("c")
    s = lax.axis_index("s")
    wid = c * NSUB + s

    pltpu.sync_copy(dst_hbm.at[wid], dstb)
    pltpu.sync_copy(src_hbm.at[wid], srcb)
    pltpu.sync_copy(et_hbm.at[wid], etb)
    pltpu.sync_copy(wt_hbm.at[wid], wtb)
    pltpu.sync_copy(nt_hbm, ntb.at[pl.ds(0, N)])
    pltpu.sync_copy(stats_hbm, statsb)
    pltpu.sync_copy(delta_hbm, deltab)

    @pl.when(s == 0)
    def _():
        pltpu.sync_copy(z2_hbm.at[0], denom_sh)
        pltpu.sync_copy(z2_hbm.at[1], pos_sh)

    plsc.subcore_barrier()

    mu = statsb[0, pl.ds(0, 16)]
    ivs = statsb[1, pl.ds(0, 16)]
    dl = deltab[...]

    def chunk(j, carry):
        def sub(k, carry2):
            sl = pl.ds(k * 16, 16)
            dstv = dstb[j, sl]
            ntv = plsc.load_gather(ntb, [dstv])
            etv = etb[j, sl]
            wv = wtb[j, sl]
            srcv = srcb[j, sl].astype(_f32)
            td = ntv - etv
            ex = jnp.exp(-dl * td)
            exb[j, sl] = ex
            pb[j, sl] = srcv * jnp.exp(jnp.abs(td))
            cb[j, sl] = (wv - mu) * ivs * ex
            return carry2

        lax.fori_loop(0, CH // 16, sub, 0)
        pltpu.sync_copy(exb.at[j], denom_sh.at[dstb.at[j]], add=True)
        pltpu.sync_copy(pb.at[j], pos_sh.at[dstb.at[j]], add=True)
        return carry

    lax.fori_loop(0, NCH, chunk, 0)
    pltpu.sync_copy(cb, c_hbm.at[wid])

    plsc.subcore_barrier()

    @pl.when(s == 0)
    def _():
        pltpu.sync_copy(denom_sh, dp_hbm.at[c, 0])
        pltpu.sync_copy(pos_sh, dp_hbm.at[c, 1])


def _passA(dstp, srcp, etp, wtp, node_time, stats, delta16, z2):
    mesh = plsc.VectorSubcoreMesh(core_axis_name="c", subcore_axis_name="s")
    k = pl.kernel(
        _passA_body,
        out_type=(jax.ShapeDtypeStruct((NW, NCH, CH), _f32),
                  jax.ShapeDtypeStruct((2, 2, N_PAD), _f32)),
        mesh=mesh,
        compiler_params=pltpu.CompilerParams(needs_layout_passes=False),
        scratch_types=[
            pltpu.VMEM((NCH, CH), _i32),
            pltpu.VMEM((NCH, CH), _i32),
            pltpu.VMEM((NCH, CH), _f32),
            pltpu.VMEM((NCH, CH), _f32),
            pltpu.VMEM((N_PAD,), _f32),
            pltpu.VMEM((8, 128), _f32),
            pltpu.VMEM((16,), _f32),
            pltpu.VMEM((NCH, CH), _f32),
            pltpu.VMEM((NCH, CH), _f32),
            pltpu.VMEM((NCH, CH), _f32),
            pltpu.VMEM_SHARED((N_PAD,), _f32),
            pltpu.VMEM_SHARED((N_PAD,), _f32),
        ],
    )
    return k(dstp, srcp, etp, wtp, node_time, stats, delta16, z2)


def _passB_body(x0_hbm, x1_hbm, src_hbm, dst_hbm, cin_hbm, dp_hbm, zh_hbm,
                *rest, with_rec):
    if with_rec:
        (out_hbm, c2out_hbm,
         srcb, dstb, cb, c2b, recb, d0b, rowsb, sem, aggr_sh) = rest
    else:
        (out_hbm,
         srcb, dstb, cb, c2b, recb, d0b, rowsb, sem, aggr_sh) = rest

    c = lax.axis_index("c")
    s = lax.axis_index("s")
    wid = c * NSUB + s

    pltpu.sync_copy(src_hbm.at[wid], srcb)
    pltpu.sync_copy(dst_hbm.at[wid], dstb)
    pltpu.sync_copy(cin_hbm.at[wid], cb)

    if with_rec:
        pltpu.sync_copy(dp_hbm.at[0, 0], d0b)
        pltpu.sync_copy(dp_hbm.at[1, 0], recb)

        def mkrec(k, carry):
            sl = pl.ds(k * 16, 16)
            v = d0b[sl] + recb[sl]
            recb[sl] = 1.0 / (v + 1e-12)
            return carry

        lax.fori_loop(0, N_PAD // 16, mkrec, 0)

        def c2chunk(j, carry):
            def c2sub(k, carry2):
                sl = pl.ds(k * 16, 16)
                dstv = dstb[j, sl]
                rv = plsc.load_gather(recb, [dstv])
                c2b[j, sl] = cb[j, sl] * rv
                return carry2
            lax.fori_loop(0, CH // 16, c2sub, 0)
            return carry

        lax.fori_loop(0, NCH, c2chunk, 0)
        pltpu.sync_copy(c2b, c2out_hbm.at[wid])

    coeff = c2b if with_rec else cb

    for h, xh_hbm in enumerate((x0_hbm, x1_hbm)):
        @pl.when(s == 0)
        def _zero():
            pltpu.sync_copy(zh_hbm, aggr_sh)

        plsc.subcore_barrier()

        def chunk(j, carry):
            pltpu.async_copy(xh_hbm.at[srcb.at[j]], rowsb, sem).wait()

            def row(r, carry2):
                cs = plsc.load_gather(
                    coeff,
                    [jnp.full((16,), j, _i32), jnp.full((16,), r, _i32)])
                for m in range(4):
                    sl = pl.ds(m * 16, 16)
                    rowsb[r, sl] = rowsb[r, sl] * cs
                return carry2

            lax.fori_loop(0, CH, row, 0)
            pltpu.sync_copy(rowsb, aggr_sh.at[dstb.at[j]], add=True)
            return carry

        lax.fori_loop(0, NCH, chunk, 0)

        plsc.subcore_barrier()

        @pl.when(s == 0)
        def _writeout():
            pltpu.sync_copy(aggr_sh, out_hbm.at[c, h])

        plsc.subcore_barrier()


def _passB(x0, x1, srcp, dstp, cin, dp, zh, with_rec):
    mesh = plsc.VectorSubcoreMesh(core_axis_name="c", subcore_axis_name="s")
    if with_rec:
        out_type = (jax.ShapeDtypeStruct((2, 2, N_PAD, D // 2), _f32),
                    jax.ShapeDtypeStruct((NW, NCH, CH), _f32))
    else:
        out_type = (jax.ShapeDtypeStruct((2, 2, N_PAD, D // 2), _f32),)
    body = functools.partial(_passB_body, with_rec=with_rec)
    k = pl.kernel(
        body,
        out_type=out_type,
        mesh=mesh,
        compiler_params=pltpu.CompilerParams(needs_layout_passes=False,
                                             use_tc_tiling_on_sc=False),
        scratch_types=[
            pltpu.VMEM((NCH, CH), _i32),
            pltpu.VMEM((NCH, CH), _i32),
            pltpu.VMEM((NCH, CH), _f32),
            pltpu.VMEM((NCH, CH), _f32),
            pltpu.VMEM((N_PAD,), _f32),
            pltpu.VMEM((N_PAD,), _f32),
            pltpu.VMEM((CH, D // 2), _f32),
            pltpu.SemaphoreType.DMA,
            pltpu.VMEM_SHARED((N_PAD, D // 2), _f32),
        ],
    )
    return k(x0, x1, srcp, dstp, cin, dp, zh)


def kernel(x, edge_index, edge_weights, edge_times, node_time, node_ids,
           delta,
           Ws1, bs1, Wh1, bh1, gamma1, beta1, Wfc1, bfc1,
           Ws2, bs2, Wh2, bh2, gamma2, beta2, Wfc2, bfc2):
    dst = edge_index[0].astype(_i32)
    src = edge_index[1].astype(_i32)
    pad = E_PAD - E

    dstp = jnp.concatenate([dst, jnp.zeros((pad,), _i32)]).reshape(NW, NCH, CH)
    srcp = jnp.concatenate([src, jnp.zeros((pad,), _i32)]).reshape(NW, NCH, CH)
    etp = jnp.concatenate(
        [edge_times, jnp.full((pad,), -60.0, _f32)]).reshape(NW, NCH, CH)
    wtp = jnp.concatenate(
        [edge_weights, jnp.zeros((pad,), _f32)]).reshape(NW, NCH, CH)
    delta16 = jnp.full((16,), delta, _f32)
    z2 = jnp.zeros((2, N_PAD), _f32)
    zh = jnp.zeros((N_PAD, D // 2), _f32)

    stats = _edge_stats(edge_weights)
    c3, dp = _passA(dstp, srcp, etp, wtp, node_time, stats, delta16, z2)

    p1, c23 = _passB(x[:, :D // 2], x[:, D // 2:], srcp, dstp, c3, dp, zh,
                     with_rec=True)
    x2, _ = _dense_layer(p1, x, dp.reshape(4, N_PAD),
                         Ws1, Wh1, bs1, bh1, gamma1, beta1, Wfc1, bfc1,
                         add_x=True, emit_pos=False)

    (p2,) = _passB(x2[:, :D // 2], x2[:, D // 2:], srcp, dstp, c23, dp, zh,
                   with_rec=False)
    out, pos2d = _dense_layer(p2, x2, dp.reshape(4, N_PAD),
                              Ws2, Wh2, bs2, bh2, gamma2, beta2, Wfc2, bfc2,
                              add_x=False, emit_pos=True)

    pos = pos2d.reshape(N_PAD)[:N]
    return out, pos
